# Initial kernel scaffold; baseline (speedup 1.0000x reference)
import jax
import jax.numpy as jnp
from jax.experimental import pallas as pl


def kernel(monthly_fee, tenure_months, plan_type, user_level, device_brand, W_fee, b_fee, W_ten, b_ten, emb_plan, emb_level, emb_brand):
    raise NotImplementedError("write your pallas kernel here")


# speedup vs baseline: 2.3234x; 2.3234x over previous
import functools

import jax
import jax.numpy as jnp
from jax import lax
from jax.experimental import pallas as pl
from jax.experimental.pallas import tpu as pltpu
from jax.experimental.pallas import tpu_sc as plsc

B = 16384
D_OUT = 128
NC = 2
NS = 16
NW = NC * NS
BPW = B // NW
CH = 128
NCH = BPW // CH
UNROLL = 8


def _encode(fee2, ten2, plan3, level3, brand3, wb, eplan, elevel, ebrand):
  mesh = plsc.VectorSubcoreMesh(core_axis_name="c", subcore_axis_name="s")

  @functools.partial(
      pl.kernel,
      mesh=mesh,
      out_type=jax.ShapeDtypeStruct((B, D_OUT), jnp.float32),
      compiler_params=pltpu.CompilerParams(use_tc_tiling_on_sc=False),
      scratch_types=[
          pltpu.VMEM((NCH, CH), jnp.int32),
          pltpu.VMEM((NCH, CH), jnp.int32),
          pltpu.VMEM((NCH, CH), jnp.int32),
          pltpu.VMEM((BPW,), jnp.float32),
          pltpu.VMEM((BPW,), jnp.float32),
          pltpu.VMEM((128,), jnp.float32),
          pltpu.VMEM((BPW, 32), jnp.float32),
          pltpu.VMEM((BPW, 16), jnp.float32),
          pltpu.VMEM((BPW, 16), jnp.float32),
          pltpu.VMEM((BPW, 64), jnp.float32),
          pltpu.SemaphoreType.DMA,
          pltpu.SemaphoreType.DMA,
      ],
  )
  def enc(fee_hbm, ten_hbm, plan_hbm, level_hbm, brand_hbm, wb_hbm,
          eplan_hbm, elevel_hbm, ebrand_hbm, out_hbm,
          pidx, lidx, bidx, fee_v, ten_v, wb_v,
          prow, lrow, brow, num_v, gsem, osem):
    wid = lax.axis_index("s") * NC + lax.axis_index("c")
    base = wid * BPW

    pltpu.sync_copy(plan_hbm.at[wid], pidx)
    pltpu.sync_copy(level_hbm.at[wid], lidx)
    pltpu.sync_copy(brand_hbm.at[wid], bidx)

    gathers = []
    for c in range(NCH):
      gathers.append(pltpu.async_copy(
          eplan_hbm.at[pidx.at[c]], prow.at[pl.ds(c * CH, CH)], gsem))
      gathers.append(pltpu.async_copy(
          elevel_hbm.at[lidx.at[c]], lrow.at[pl.ds(c * CH, CH)], gsem))
      gathers.append(pltpu.async_copy(
          ebrand_hbm.at[bidx.at[c]], brow.at[pl.ds(c * CH, CH)], gsem))

    pltpu.sync_copy(fee_hbm.at[wid], fee_v)
    pltpu.sync_copy(ten_hbm.at[wid], ten_v)
    pltpu.sync_copy(wb_hbm, wb_v)

    wf_lo = wb_v[pl.ds(0, 16)]
    wf_hi = wb_v[pl.ds(16, 16)]
    bf_lo = wb_v[pl.ds(32, 16)]
    bf_hi = wb_v[pl.ds(48, 16)]
    wt_lo = wb_v[pl.ds(64, 16)]
    wt_hi = wb_v[pl.ds(80, 16)]
    bt_lo = wb_v[pl.ds(96, 16)]
    bt_hi = wb_v[pl.ds(112, 16)]

    def rows(g, carry):
      r0 = g * 16
      f16 = fee_v[pl.ds(r0, 16)]
      t16 = ten_v[pl.ds(r0, 16)]
      for k in range(16):
        i = r0 + k
        f = jnp.full((16,), f16[k], jnp.float32)
        t = jnp.full((16,), t16[k], jnp.float32)
        num_v[i, pl.ds(0, 16)] = f * wf_lo + bf_lo
        num_v[i, pl.ds(16, 16)] = f * wf_hi + bf_hi
        num_v[i, pl.ds(32, 16)] = t * wt_lo + bt_lo
        num_v[i, pl.ds(48, 16)] = t * wt_hi + bt_hi
      return carry

    lax.fori_loop(0, BPW // 16, rows, 0)

    out_num = pltpu.async_copy(
        num_v, out_hbm.at[pl.ds(base, BPW), pl.ds(0, 64)], osem)
    for g in gathers:
      g.wait()
    out_p = pltpu.async_copy(
        prow, out_hbm.at[pl.ds(base, BPW), pl.ds(64, 32)], osem)
    out_l = pltpu.async_copy(
        lrow, out_hbm.at[pl.ds(base, BPW), pl.ds(96, 16)], osem)
    out_b = pltpu.async_copy(
        brow, out_hbm.at[pl.ds(base, BPW), pl.ds(112, 16)], osem)
    out_num.wait()
    out_p.wait()
    out_l.wait()
    out_b.wait()

  return enc(fee2, ten2, plan3, level3, brand3, wb, eplan, elevel, ebrand)


def kernel(monthly_fee, tenure_months, plan_type, user_level, device_brand,
           W_fee, b_fee, W_ten, b_ten, emb_plan, emb_level, emb_brand):
  fee2 = monthly_fee.reshape(NW, BPW)
  ten2 = tenure_months.reshape(NW, BPW)
  plan3 = plan_type.reshape(NW, NCH, CH)
  level3 = user_level.reshape(NW, NCH, CH)
  brand3 = device_brand.reshape(NW, NCH, CH)
  wb = jnp.concatenate(
      [W_fee.reshape(32), b_fee, W_ten.reshape(32), b_ten])
  return _encode(fee2, ten2, plan3, level3, brand3, wb,
                 emb_plan, emb_level, emb_brand)


# speedup vs baseline: 2.3714x; 1.0206x over previous
import functools

import jax
import jax.numpy as jnp
from jax import lax
from jax.experimental import pallas as pl
from jax.experimental.pallas import tpu as pltpu
from jax.experimental.pallas import tpu_sc as plsc

B = 16384
D_OUT = 128
NC = 2
NS = 16
NW = NC * NS
BPW = B // NW
CH = 128
NCH = BPW // CH
UNROLL = 8


def _encode(fee2, ten2, plan3, level3, brand3, wb, eplan, elevel, ebrand):
  mesh = plsc.VectorSubcoreMesh(core_axis_name="c", subcore_axis_name="s")

  @functools.partial(
      pl.kernel,
      mesh=mesh,
      out_type=jax.ShapeDtypeStruct((B, D_OUT), jnp.float32),
      compiler_params=pltpu.CompilerParams(use_tc_tiling_on_sc=False),
      scratch_types=[
          pltpu.VMEM((NCH, CH), jnp.int32),
          pltpu.VMEM((NCH, CH), jnp.int32),
          pltpu.VMEM((NCH, CH), jnp.int32),
          pltpu.VMEM((BPW,), jnp.float32),
          pltpu.VMEM((BPW,), jnp.float32),
          pltpu.VMEM((128,), jnp.float32),
          pltpu.VMEM((BPW, 32), jnp.float32),
          pltpu.VMEM((BPW, 16), jnp.float32),
          pltpu.VMEM((BPW, 16), jnp.float32),
          pltpu.VMEM((BPW, 64), jnp.float32),
          pltpu.SemaphoreType.DMA,
          pltpu.SemaphoreType.DMA,
      ],
  )
  def enc(fee_hbm, ten_hbm, plan_hbm, level_hbm, brand_hbm, wb_hbm,
          eplan_hbm, elevel_hbm, ebrand_hbm, out_hbm,
          pidx, lidx, bidx, fee_v, ten_v, wb_v,
          prow, lrow, brow, num_v, gsem, osem):
    wid = lax.axis_index("s") * NC + lax.axis_index("c")
    base = wid * BPW

    stage = [
        pltpu.async_copy(plan_hbm.at[wid], pidx, osem),
        pltpu.async_copy(level_hbm.at[wid], lidx, osem),
        pltpu.async_copy(brand_hbm.at[wid], bidx, osem),
        pltpu.async_copy(fee_hbm.at[wid], fee_v, osem),
        pltpu.async_copy(ten_hbm.at[wid], ten_v, osem),
        pltpu.async_copy(wb_hbm, wb_v, osem),
    ]
    for s in stage[:3]:
      s.wait()

    gathers = []
    for c in range(NCH):
      gathers.append(pltpu.async_copy(
          eplan_hbm.at[pidx.at[c]], prow.at[pl.ds(c * CH, CH)], gsem))
      gathers.append(pltpu.async_copy(
          elevel_hbm.at[lidx.at[c]], lrow.at[pl.ds(c * CH, CH)], gsem))
      gathers.append(pltpu.async_copy(
          ebrand_hbm.at[bidx.at[c]], brow.at[pl.ds(c * CH, CH)], gsem))

    for s in stage[3:]:
      s.wait()

    wf_lo = wb_v[pl.ds(0, 16)]
    wf_hi = wb_v[pl.ds(16, 16)]
    bf_lo = wb_v[pl.ds(32, 16)]
    bf_hi = wb_v[pl.ds(48, 16)]
    wt_lo = wb_v[pl.ds(64, 16)]
    wt_hi = wb_v[pl.ds(80, 16)]
    bt_lo = wb_v[pl.ds(96, 16)]
    bt_hi = wb_v[pl.ds(112, 16)]

    def rows(g, carry):
      r0 = g * 16
      f16 = fee_v[pl.ds(r0, 16)]
      t16 = ten_v[pl.ds(r0, 16)]
      for k in range(16):
        i = r0 + k
        f = jnp.full((16,), f16[k], jnp.float32)
        t = jnp.full((16,), t16[k], jnp.float32)
        num_v[i, pl.ds(0, 16)] = f * wf_lo + bf_lo
        num_v[i, pl.ds(16, 16)] = f * wf_hi + bf_hi
        num_v[i, pl.ds(32, 16)] = t * wt_lo + bt_lo
        num_v[i, pl.ds(48, 16)] = t * wt_hi + bt_hi
      return carry

    lax.fori_loop(0, BPW // 16, rows, 0)

    out_num = pltpu.async_copy(
        num_v, out_hbm.at[pl.ds(base, BPW), pl.ds(0, 64)], osem)
    for g in gathers:
      g.wait()
    out_p = pltpu.async_copy(
        prow, out_hbm.at[pl.ds(base, BPW), pl.ds(64, 32)], osem)
    out_l = pltpu.async_copy(
        lrow, out_hbm.at[pl.ds(base, BPW), pl.ds(96, 16)], osem)
    out_b = pltpu.async_copy(
        brow, out_hbm.at[pl.ds(base, BPW), pl.ds(112, 16)], osem)
    out_num.wait()
    out_p.wait()
    out_l.wait()
    out_b.wait()

  return enc(fee2, ten2, plan3, level3, brand3, wb, eplan, elevel, ebrand)


def kernel(monthly_fee, tenure_months, plan_type, user_level, device_brand,
           W_fee, b_fee, W_ten, b_ten, emb_plan, emb_level, emb_brand):
  fee2 = monthly_fee.reshape(NW, BPW)
  ten2 = tenure_months.reshape(NW, BPW)
  plan3 = plan_type.reshape(NW, NCH, CH)
  level3 = user_level.reshape(NW, NCH, CH)
  brand3 = device_brand.reshape(NW, NCH, CH)
  wb = jnp.concatenate(
      [W_fee.reshape(32), b_fee, W_ten.reshape(32), b_ten])
  return _encode(fee2, ten2, plan3, level3, brand3, wb,
                 emb_plan, emb_level, emb_brand)
